# baseline (speedup 1.0000x reference)
import functools

import jax
import jax.numpy as jnp
from jax import lax
from jax.experimental import pallas as pl
from jax.experimental.pallas import tpu as pltpu
from jax.experimental.pallas import tpu_sc as plsc

N_TRACKS = 1000000
DIM = 64
BATCH = 4096
HIST = 50

NC = 2
NS = 16
NW = NC * NS

NSLAB_FULL = N_TRACKS // 128
TAIL = N_TRACKS - NSLAB_FULL * 128
SLAB_BASE = NSLAB_FULL // NW
SLAB_REM = NSLAB_FULL % NW
WIDE = N_TRACKS // 2


def _transpose_body(tt_hbm, tail_hbm, wide_hbm, in_v, out_v, isem, osem):
    wid = lax.axis_index("s") * NC + lax.axis_index("c")
    n_s = SLAB_BASE + jnp.where(wid < SLAB_REM, 1, 0)
    start = wid * SLAB_BASE + jnp.minimum(wid, SLAB_REM)

    def do_slab(col, out_row, p):
        pltpu.async_copy(
            tt_hbm.at[:, pl.ds(pl.multiple_of(col, 128), 128)], in_v.at[p],
            isem,
        ).wait()

        @pl.loop(0, 128, step=8)
        def _t(t0):
            for dt in range(8):
                t = t0 + dt
                row = t >> 1
                half = (t & 1) * 64
                for c in range(4):
                    rows_c = c * 16 + lax.iota(jnp.int32, 16)
                    cols_t = jnp.zeros((16,), jnp.int32) + t
                    vals = plsc.load_gather(in_v.at[p], [rows_c, cols_t])
                    out_v[p, row, pl.ds(half + c * 16, 16)] = vals

        pltpu.async_copy(
            out_v.at[p], wide_hbm.at[pl.ds(pl.multiple_of(out_row, 8), 64)],
            osem,
        )

    @pl.loop(0, SLAB_BASE + 1, step=2)
    def _pair(i):
        for p in range(2):
            k = i + p

            @pl.when(k < n_s)
            def _go():
                @pl.when(k >= 2)
                def _drain():
                    pltpu.make_async_copy(
                        out_v.at[p], wide_hbm.at[pl.ds(0, 64)], osem
                    ).wait()

                s = start + k
                do_slab(s * 128, s * 64, p)

    for _ in range(2):
        pltpu.make_async_copy(
            out_v.at[0], wide_hbm.at[pl.ds(0, 64)], osem
        ).wait()

    @pl.when(wid == NW - 1)
    def _tail():
        pltpu.async_copy(
            tail_hbm, in_v.at[0, pl.ds(0, TAIL // 2)], isem
        ).wait()
        pltpu.async_copy(
            in_v.at[0, pl.ds(0, TAIL // 2)],
            wide_hbm.at[pl.ds(WIDE - TAIL // 2, TAIL // 2)],
            osem,
        ).wait()


def _gather_body(idxt_hbm, wide_hbm, out_hbm, idx_v, gidx_v, hbase_v,
                 staged_v, outt_v, isem, gsem, osem):
    wid = lax.axis_index("s") * NC + lax.axis_index("c")

    pltpu.async_copy(
        idxt_hbm.at[:, pl.ds(pl.multiple_of(wid * 128, 128), 128)], idx_v,
        isem,
    ).wait()

    def fire(h, p):
        for k in range(8):
            v = idx_v[h, pl.ds(k * 16, 16)]
            gidx_v[p, pl.ds(k * 16, 16)] = v >> 1
            hbase_v[p, pl.ds(k * 16, 16)] = (v & 1) * 64
        pltpu.async_copy(
            wide_hbm.at[gidx_v.at[p]], staged_v.at[p], gsem
        )

    def process(h, p):
        pltpu.make_async_copy(
            wide_hbm.at[gidx_v.at[p]], staged_v.at[p], gsem
        ).wait()

        @pl.when(h >= 2)
        def _drain():
            pltpu.make_async_copy(
                outt_v.at[p], out_hbm.at[0, :, pl.ds(0, 128)], osem
            ).wait()

        @pl.loop(0, DIM)
        def _d(d):
            for k in range(8):
                rows = k * 16 + lax.iota(jnp.int32, 16)
                cols = hbase_v[p, pl.ds(k * 16, 16)] + d
                vals = plsc.load_gather(staged_v.at[p], [rows, cols])
                outt_v[p, d, pl.ds(k * 16, 16)] = vals

        pltpu.async_copy(
            outt_v.at[p],
            out_hbm.at[h, :, pl.ds(pl.multiple_of(wid * 128, 128), 128)],
            osem,
        )

    fire(0, 0)
    fire(1, 1)

    @pl.loop(0, HIST, step=2)
    def _h(h0):
        for p in range(2):
            h = h0 + p
            process(h, p)

            @pl.when(h + 2 < HIST)
            def _f2():
                fire(h + 2, p)

    for p in range(2):
        pltpu.make_async_copy(
            outt_v.at[p], out_hbm.at[0, :, pl.ds(0, 128)], osem
        ).wait()


@jax.jit
def kernel(indices_, table):
    tt = table.T
    idxt = indices_.T
    tail_wide = table[N_TRACKS - TAIL:].reshape(TAIL // 2, 128)
    mesh = plsc.VectorSubcoreMesh(
        core_axis_name="c", subcore_axis_name="s", num_cores=NC, num_subcores=NS
    )
    wide = pl.kernel(
        _transpose_body,
        out_type=jax.ShapeDtypeStruct((WIDE, 128), jnp.float32),
        mesh=mesh,
        scratch_types=[
            pltpu.VMEM((2, 64, 128), jnp.float32),
            pltpu.VMEM((2, 64, 128), jnp.float32),
            pltpu.SemaphoreType.DMA,
            pltpu.SemaphoreType.DMA,
        ],
        compiler_params=pltpu.CompilerParams(needs_layout_passes=False),
    )(tt, tail_wide)
    out3 = pl.kernel(
        _gather_body,
        out_type=jax.ShapeDtypeStruct((HIST, DIM, BATCH), jnp.float32),
        mesh=mesh,
        scratch_types=[
            pltpu.VMEM((HIST, 128), jnp.int32),
            pltpu.VMEM((2, 128), jnp.int32),
            pltpu.VMEM((2, 128), jnp.int32),
            pltpu.VMEM((2, 128, 128), jnp.float32),
            pltpu.VMEM((2, DIM, 128), jnp.float32),
            pltpu.SemaphoreType.DMA,
            pltpu.SemaphoreType.DMA,
            pltpu.SemaphoreType.DMA,
        ],
        compiler_params=pltpu.CompilerParams(needs_layout_passes=False),
    )(idxt, wide)
    return jnp.transpose(out3, (2, 0, 1))


# speedup vs baseline: 2.8883x; 2.8883x over previous
import jax
import jax.numpy as jnp
from jax import lax
from jax.experimental import pallas as pl
from jax.experimental.pallas import tpu as pltpu
from jax.experimental.pallas import tpu_sc as plsc

N_TRACKS = 1000000
DIM = 64
BATCH = 4096
HIST = 50

NC = 2
NS = 16
NW = NC * NS
FLAT = BATCH * HIST
PER_W = FLAT // NW
CHUNK = 128
NCHUNK = PER_W // CHUNK
NBUF = 5


def _gather_body(idx_hbm, table_hbm, out_hbm, idx_v, bufs, gsem, osem):
    wid = lax.axis_index("s") * NC + lax.axis_index("c")
    base = wid * PER_W

    pltpu.sync_copy(idx_hbm.at[pl.ds(pl.multiple_of(wid * NCHUNK, NCHUNK),
                                     NCHUNK)], idx_v)

    def fire(c, b):
        pltpu.async_copy(table_hbm.at[idx_v.at[c]], bufs.at[b], gsem.at[b])

    for c in range(NBUF):
        fire(c, c)

    @pl.loop(0, NCHUNK, step=NBUF)
    def _ring(c0):
        for b in range(NBUF):
            c = c0 + b
            pltpu.make_async_copy(
                table_hbm.at[idx_v.at[c]], bufs.at[b], gsem.at[b]
            ).wait()
            pltpu.async_copy(
                bufs.at[b],
                out_hbm.at[pl.ds(pl.multiple_of(base + c * CHUNK, CHUNK),
                                 CHUNK)],
                osem.at[b],
            )

            @pl.when(c + NBUF < NCHUNK)
            def _refill():
                pltpu.make_async_copy(
                    bufs.at[b], out_hbm.at[pl.ds(0, CHUNK)], osem.at[b]
                ).wait()
                fire(c + NBUF, b)

    for b in range(NBUF):
        pltpu.make_async_copy(
            bufs.at[b], out_hbm.at[pl.ds(0, CHUNK)], osem.at[b]
        ).wait()


@jax.jit
def kernel(indices_, table):
    idx2 = indices_.reshape(FLAT // CHUNK, CHUNK)
    mesh = plsc.VectorSubcoreMesh(
        core_axis_name="c", subcore_axis_name="s", num_cores=NC, num_subcores=NS
    )
    out = pl.kernel(
        _gather_body,
        out_type=jax.ShapeDtypeStruct((FLAT, DIM), jnp.float32),
        mesh=mesh,
        scratch_types=[
            pltpu.VMEM((NCHUNK, CHUNK), jnp.int32),
            pltpu.VMEM((NBUF, CHUNK, DIM), jnp.float32),
            pltpu.SemaphoreType.DMA((NBUF,)),
            pltpu.SemaphoreType.DMA((NBUF,)),
        ],
        compiler_params=pltpu.CompilerParams(use_tc_tiling_on_sc=False),
    )(idx2, table)
    return out.reshape(BATCH, HIST, DIM)


# speedup vs baseline: 2.8930x; 1.0016x over previous
import jax
import jax.numpy as jnp
from jax import lax
from jax.experimental import pallas as pl
from jax.experimental.pallas import tpu as pltpu
from jax.experimental.pallas import tpu_sc as plsc

N_TRACKS = 1000000
DIM = 64
BATCH = 4096
HIST = 50

NC = 2
NS = 16
NW = NC * NS
FLAT = BATCH * HIST
PER_W = FLAT // NW
CHUNK = 128
NCHUNK = PER_W // CHUNK
NBUF = 10


def _gather_body(idx_hbm, table_hbm, out_hbm, idx_v, bufs, gsem, osem):
    wid = lax.axis_index("s") * NC + lax.axis_index("c")
    base = wid * PER_W

    pltpu.sync_copy(idx_hbm.at[pl.ds(pl.multiple_of(wid * NCHUNK, NCHUNK),
                                     NCHUNK)], idx_v)

    def fire(c, b):
        pltpu.async_copy(table_hbm.at[idx_v.at[c]], bufs.at[b], gsem.at[b])

    for c in range(NBUF):
        fire(c, c)

    @pl.loop(0, NCHUNK, step=NBUF)
    def _ring(c0):
        for b in range(NBUF):
            c = c0 + b
            pltpu.make_async_copy(
                table_hbm.at[idx_v.at[c]], bufs.at[b], gsem.at[b]
            ).wait()
            pltpu.async_copy(
                bufs.at[b],
                out_hbm.at[pl.ds(pl.multiple_of(base + c * CHUNK, CHUNK),
                                 CHUNK)],
                osem.at[b],
            )

            @pl.when(c + NBUF < NCHUNK)
            def _refill():
                pltpu.make_async_copy(
                    bufs.at[b], out_hbm.at[pl.ds(0, CHUNK)], osem.at[b]
                ).wait()
                fire(c + NBUF, b)

    for b in range(NBUF):
        pltpu.make_async_copy(
            bufs.at[b], out_hbm.at[pl.ds(0, CHUNK)], osem.at[b]
        ).wait()


@jax.jit
def kernel(indices_, table):
    idx2 = indices_.reshape(FLAT // CHUNK, CHUNK)
    mesh = plsc.VectorSubcoreMesh(
        core_axis_name="c", subcore_axis_name="s", num_cores=NC, num_subcores=NS
    )
    out = pl.kernel(
        _gather_body,
        out_type=jax.ShapeDtypeStruct((FLAT, DIM), jnp.float32),
        mesh=mesh,
        scratch_types=[
            pltpu.VMEM((NCHUNK, CHUNK), jnp.int32),
            pltpu.VMEM((NBUF, CHUNK, DIM), jnp.float32),
            pltpu.SemaphoreType.DMA((NBUF,)),
            pltpu.SemaphoreType.DMA((NBUF,)),
        ],
        compiler_params=pltpu.CompilerParams(use_tc_tiling_on_sc=False),
    )(idx2, table)
    return out.reshape(BATCH, HIST, DIM)
